# Initial kernel scaffold; baseline (speedup 1.0000x reference)
import jax
import jax.numpy as jnp
from jax.experimental import pallas as pl


def kernel(sequence_tensor, positions):
    raise NotImplementedError("write your pallas kernel here")


# speedup vs baseline: 1.3971x; 1.3971x over previous
import functools

import jax
import jax.numpy as jnp
from jax import lax
from jax.experimental import pallas as pl
from jax.experimental.pallas import tpu as pltpu
from jax.experimental.pallas import tpu_sc as plsc


def _gather_call(n_rows, width, seq_len, rows_per_batch):
    info = plsc.get_sparse_core_info()
    nc, ns, lanes = info.num_cores, info.num_subcores, info.num_lanes
    nw = nc * ns
    assert n_rows % nw == 0
    per_w = n_rows // nw
    assert per_w % lanes == 0 and rows_per_batch % per_w == 0
    mesh = plsc.VectorSubcoreMesh(core_axis_name="c", subcore_axis_name="s")

    @functools.partial(
        pl.kernel,
        mesh=mesh,
        out_type=jax.ShapeDtypeStruct((n_rows, width), jnp.float32),
        scratch_types=[
            pltpu.VMEM((per_w,), jnp.int32),
            pltpu.VMEM((per_w, width), jnp.float32),
            pltpu.SemaphoreType.DMA,
        ],
    )
    def k(table_hbm, pos_hbm, out_hbm, idx_v, rows_v, sem):
        wid = lax.axis_index("s") * nc + lax.axis_index("c")
        base = wid * per_w
        pltpu.sync_copy(pos_hbm.at[pl.ds(base, per_w)], idx_v)
        off = (base // rows_per_batch) * seq_len
        for i in range(per_w // lanes):
            sl = pl.ds(i * lanes, lanes)
            idx_v[sl] = idx_v[sl] + off
        pltpu.async_copy(table_hbm.at[idx_v], rows_v, sem).wait()
        pltpu.sync_copy(rows_v, out_hbm.at[pl.ds(base, per_w)])

    return k


def kernel(sequence_tensor, positions):
    batch, seq_len, width = sequence_tensor.shape
    n_rows = positions.shape[0] * positions.shape[1]
    table = sequence_tensor.reshape(batch * seq_len, width)
    flat_pos = positions.reshape(n_rows).astype(jnp.int32)
    call = _gather_call(n_rows, width, seq_len, positions.shape[1])
    return call(table, flat_pos)
